# Initial kernel scaffold; baseline (speedup 1.0000x reference)
import jax
import jax.numpy as jnp
from jax.experimental import pallas as pl


def kernel(seq_len, table):
    raise NotImplementedError("write your pallas kernel here")


# speedup vs baseline: 6.4845x; 6.4845x over previous
import functools

import jax
import jax.numpy as jnp
from jax import lax
from jax.experimental import pallas as pl
from jax.experimental.pallas import tpu as pltpu
from jax.experimental.pallas import tpu_sc as plsc

_NUM_HEADS = 16
_MAX_DIST = 128
_S = 2048
_HALF = _S // 2
_NSHIFT = 8
_UM = 3200
_L = 16
_GROUP = 16
_GGROUP = 8


def _sc_body(table_hbm, out_hbm, idx_v, m_v, sem):
    h = lax.axis_index("s")
    half = lax.axis_index("c")
    r0 = half * _HALF
    b_base = r0 + _HALF + 6

    lane = lax.iota(jnp.int32, _L)

    def build_idx(c, _):
        u0 = c * _L
        for s in range(_NSHIFT):
            d = u0 + lane - b_base + s
            idx = (jnp.clip(d, -_MAX_DIST, _MAX_DIST) + _MAX_DIST) * _NUM_HEADS + h
            idx_v[pl.ds(pl.multiple_of(s * _UM + u0, _L), _L)] = idx
        return _

    lax.fori_loop(0, _UM // _L, build_idx, None)

    def gather_group(q, _):
        handles = []
        for k in range(_GGROUP):
            off = pl.multiple_of((q * _GGROUP + k) * 128, 128)
            handles.append(
                pltpu.async_copy(
                    table_hbm.at[idx_v.at[pl.ds(off, 128)]],
                    m_v.at[pl.ds(off, 128)],
                    sem,
                )
            )
        for hd in handles:
            hd.wait()
        return _

    lax.fori_loop(0, _NSHIFT * _UM // (128 * _GGROUP), gather_group, None)

    def row_group(g, _):
        base = r0 + g * _GROUP
        handles = []
        for k in range(_GROUP):
            i = base + k
            t = b_base - i
            s = lax.rem(t, _NSHIFT)
            o = pl.multiple_of(s * _UM + (t - s), _NSHIFT)
            dst = pl.multiple_of((h * _S + i) * _S, _S)
            handles.append(
                pltpu.async_copy(
                    m_v.at[pl.ds(o, _S)], out_hbm.at[pl.ds(dst, _S)], sem
                )
            )
        for hd in handles:
            hd.wait()
        return _

    lax.fori_loop(0, _HALF // _GROUP, row_group, None)


@functools.partial(
    pl.kernel,
    mesh=plsc.VectorSubcoreMesh(core_axis_name="c", subcore_axis_name="s"),
    out_type=jax.ShapeDtypeStruct((_NUM_HEADS * _S * _S,), jnp.float32),
    scratch_types=[
        pltpu.VMEM((_NSHIFT * _UM,), jnp.int32),
        pltpu.VMEM((_NSHIFT * _UM,), jnp.float32),
        pltpu.SemaphoreType.DMA,
    ],
)
def _sc_rel_pos(table_hbm, out_hbm, idx_v, m_v, sem):
    _sc_body(table_hbm, out_hbm, idx_v, m_v, sem)


def kernel(seq_len, table):
    del seq_len
    out = _sc_rel_pos(table.reshape(-1))
    return out.reshape(_NUM_HEADS, _S, _S)
